# Initial kernel scaffold; baseline (speedup 1.0000x reference)
import jax
import jax.numpy as jnp
from jax.experimental import pallas as pl


def kernel(pred, target):
    raise NotImplementedError("write your pallas kernel here")


# speedup vs baseline: 89.8713x; 89.8713x over previous
import functools

import jax
import jax.numpy as jnp
from jax import lax
from jax.experimental import pallas as pl
from jax.experimental.pallas import tpu as pltpu
from jax.experimental.pallas import tpu_sc as plsc

_B = 16
_C = 19
_HW = 512 * 512
_N = _B * _HW

_NC = 2
_NS = 16
_NW = _NC * _NS
_PER_W = _N // _NW
_CHUNK = 16384
_NCHUNK = _PER_W // _CHUNK
_UNROLL = 8


def _sc_body(tgt_hbm, out_hbm, buf0, buf1, acc_v, sem0, sem1):
    wid = lax.axis_index("s") * _NC + lax.axis_index("c")
    half = wid // _B
    img = wid % _B
    base = (img * 2 + half) * _PER_W

    bufs = (buf0, buf1)
    sems = (sem0, sem1)
    copies = [None, None]
    copies[0] = pltpu.async_copy(tgt_hbm.at[pl.ds(base, _CHUNK)], buf0, sem0)

    accs = tuple(jnp.zeros((16,), jnp.int32) for _ in range(_UNROLL))
    one = jnp.full((16,), 1, jnp.int32)
    for g in range(_NCHUNK):
        if g + 1 < _NCHUNK:
            copies[(g + 1) % 2] = pltpu.async_copy(
                tgt_hbm.at[pl.ds(base + (g + 1) * _CHUNK, _CHUNK)],
                bufs[(g + 1) % 2], sems[(g + 1) % 2])
        copies[g % 2].wait()
        buf = bufs[g % 2]

        def inner(i, a, buf=buf):
            o = i * (16 * _UNROLL)
            return tuple(
                a[j] | (one << buf[pl.ds(o + j * 16, 16)])
                for j in range(_UNROLL))

        accs = lax.fori_loop(0, _CHUNK // (16 * _UNROLL), inner, accs)

    acc = accs[0]
    for j in range(1, _UNROLL):
        acc = acc | accs[j]
    acc_v[...] = acc
    pltpu.sync_copy(acc_v, out_hbm.at[half, img])


@jax.jit
def _sc_masks(flat):
    mesh = plsc.VectorSubcoreMesh(core_axis_name="c", subcore_axis_name="s")
    f = functools.partial(
        pl.kernel,
        mesh=mesh,
        out_type=jax.ShapeDtypeStruct((2, _B, 16), jnp.int32),
        scratch_types=[
            pltpu.VMEM((_CHUNK,), jnp.int32),
            pltpu.VMEM((_CHUNK,), jnp.int32),
            pltpu.VMEM((16,), jnp.int32),
            pltpu.SemaphoreType.DMA,
            pltpu.SemaphoreType.DMA,
        ],
    )(_sc_body)
    return f(flat)


def _bce_body(pred_ref, masks_ref, out_ref):
    m = masks_ref[0] | masks_ref[1]
    cls = lax.broadcasted_iota(jnp.int32, (_B, 16, _C), 2)
    bits = (m[:, :, None] >> cls) & 1
    tvect = jnp.max(bits, axis=1).astype(jnp.float32)
    x = pred_ref[...]
    p = jax.nn.sigmoid(x)
    logp = jnp.maximum(jnp.log(p), -100.0)
    log1mp = jnp.maximum(jnp.log1p(-p), -100.0)
    loss = -jnp.mean(tvect * logp + (1.0 - tvect) * log1mp)
    out_ref[...] = jnp.reshape(loss, (1, 1))


@jax.jit
def _bce(pred, masks):
    return pl.pallas_call(
        _bce_body,
        out_shape=jax.ShapeDtypeStruct((1, 1), jnp.float32),
    )(pred, masks)


def kernel(pred, target):
    flat = target.reshape(-1).astype(jnp.int32)
    masks = _sc_masks(flat)
    return _bce(pred.astype(jnp.float32), masks)[0, 0]


# speedup vs baseline: 134.6634x; 1.4984x over previous
import functools

import jax
import jax.numpy as jnp
from jax import lax
from jax.experimental import pallas as pl
from jax.experimental.pallas import tpu as pltpu
from jax.experimental.pallas import tpu_sc as plsc

_B = 16
_C = 19
_NC = 2
_ROWS = 32
_NCHUNK = 256 // _ROWS


def _sc_body(tgt_hbm, out_hbm, buf0, buf1, acc_v, sem0, sem1):
    wid = lax.axis_index("s") * _NC + lax.axis_index("c")
    half = wid // _B
    img = wid % _B
    r0 = half * 256

    bufs = (buf0, buf1)
    sems = (sem0, sem1)
    copies = [None, None]
    copies[0] = pltpu.async_copy(
        tgt_hbm.at[img, pl.ds(r0, _ROWS)], buf0, sem0)

    accs = tuple(jnp.zeros((16,), jnp.int32) for _ in range(8))
    one = jnp.full((16,), 1, jnp.int32)
    for g in range(_NCHUNK):
        if g + 1 < _NCHUNK:
            copies[(g + 1) % 2] = pltpu.async_copy(
                tgt_hbm.at[img, pl.ds(r0 + (g + 1) * _ROWS, _ROWS)],
                bufs[(g + 1) % 2], sems[(g + 1) % 2])
        copies[g % 2].wait()
        buf = bufs[g % 2]

        def inner(i, a, buf=buf):
            new = list(a)
            for j in range(32):
                new[j % 8] = new[j % 8] | (one << buf[i, pl.ds(j * 16, 16)])
            return tuple(new)

        accs = lax.fori_loop(0, _ROWS, inner, accs)

    acc = accs[0]
    for j in range(1, 8):
        acc = acc | accs[j]
    acc_v[...] = acc
    pltpu.sync_copy(acc_v, out_hbm.at[half, img])


@jax.jit
def _sc_masks(tgt):
    mesh = plsc.VectorSubcoreMesh(core_axis_name="c", subcore_axis_name="s")
    f = functools.partial(
        pl.kernel,
        mesh=mesh,
        out_type=jax.ShapeDtypeStruct((2, _B, 16), jnp.int32),
        scratch_types=[
            pltpu.VMEM((_ROWS, 512), jnp.int32),
            pltpu.VMEM((_ROWS, 512), jnp.int32),
            pltpu.VMEM((16,), jnp.int32),
            pltpu.SemaphoreType.DMA,
            pltpu.SemaphoreType.DMA,
        ],
    )(_sc_body)
    return f(tgt)


def _bce_body(pred_ref, masks_ref, out_ref):
    m = masks_ref[0] | masks_ref[1]
    cls = lax.broadcasted_iota(jnp.int32, (_B, 16, _C), 2)
    bits = (m[:, :, None] >> cls) & 1
    tvect = jnp.max(bits, axis=1).astype(jnp.float32)
    x = pred_ref[...]
    p = jax.nn.sigmoid(x)
    logp = jnp.maximum(jnp.log(p), -100.0)
    log1mp = jnp.maximum(jnp.log1p(-p), -100.0)
    loss = -jnp.mean(tvect * logp + (1.0 - tvect) * log1mp)
    out_ref[...] = jnp.reshape(loss, (1, 1))


@jax.jit
def _bce(pred, masks):
    return pl.pallas_call(
        _bce_body,
        out_shape=jax.ShapeDtypeStruct((1, 1), jnp.float32),
    )(pred, masks)


def kernel(pred, target):
    masks = _sc_masks(target.astype(jnp.int32))
    return _bce(pred.astype(jnp.float32), masks)[0, 0]


# speedup vs baseline: 139.8850x; 1.0388x over previous
import functools

import jax
import jax.numpy as jnp
from jax import lax
from jax.experimental import pallas as pl
from jax.experimental.pallas import tpu as pltpu
from jax.experimental.pallas import tpu_sc as plsc

_B = 16
_C = 19
_NC = 2
_ROWS = 32
_NCHUNK = 256 // _ROWS


_NBUF = 4


def _sc_body(tgt_hbm, out_hbm, buf0, buf1, buf2, buf3, acc_v,
             sem0, sem1, sem2, sem3):
    wid = lax.axis_index("s") * _NC + lax.axis_index("c")
    half = wid // _B
    img = wid % _B
    r0 = half * 256

    bufs = (buf0, buf1, buf2, buf3)
    sems = (sem0, sem1, sem2, sem3)
    copies = [None] * _NBUF
    for g in range(_NBUF - 1):
        copies[g] = pltpu.async_copy(
            tgt_hbm.at[img, pl.ds(r0 + g * _ROWS, _ROWS)], bufs[g], sems[g])

    accs = tuple(jnp.zeros((16,), jnp.int32) for _ in range(8))
    one = jnp.full((16,), 1, jnp.int32)
    for g in range(_NCHUNK):
        if g + _NBUF - 1 < _NCHUNK:
            copies[(g + _NBUF - 1) % _NBUF] = pltpu.async_copy(
                tgt_hbm.at[img, pl.ds(r0 + (g + _NBUF - 1) * _ROWS, _ROWS)],
                bufs[(g + _NBUF - 1) % _NBUF], sems[(g + _NBUF - 1) % _NBUF])
        copies[g % _NBUF].wait()
        buf = bufs[g % _NBUF]

        def inner(i, a, buf=buf):
            new = list(a)
            for j in range(32):
                new[j % 8] = new[j % 8] | (one << buf[i, pl.ds(j * 16, 16)])
            return tuple(new)

        accs = lax.fori_loop(0, _ROWS, inner, accs)

    acc = accs[0]
    for j in range(1, 8):
        acc = acc | accs[j]
    acc_v[...] = acc
    pltpu.sync_copy(acc_v, out_hbm.at[half, img])


@jax.jit
def _sc_masks(tgt):
    mesh = plsc.VectorSubcoreMesh(core_axis_name="c", subcore_axis_name="s")
    f = functools.partial(
        pl.kernel,
        mesh=mesh,
        out_type=jax.ShapeDtypeStruct((2, _B, 16), jnp.int32),
        scratch_types=[
            pltpu.VMEM((_ROWS, 512), jnp.int32),
            pltpu.VMEM((_ROWS, 512), jnp.int32),
            pltpu.VMEM((_ROWS, 512), jnp.int32),
            pltpu.VMEM((_ROWS, 512), jnp.int32),
            pltpu.VMEM((16,), jnp.int32),
            pltpu.SemaphoreType.DMA,
            pltpu.SemaphoreType.DMA,
            pltpu.SemaphoreType.DMA,
            pltpu.SemaphoreType.DMA,
        ],
    )(_sc_body)
    return f(tgt)


def _bce_body(pred_ref, masks_ref, out_ref):
    m = masks_ref[0] | masks_ref[1]
    cls = lax.broadcasted_iota(jnp.int32, (_B, 16, _C), 2)
    bits = (m[:, :, None] >> cls) & 1
    tvect = jnp.max(bits, axis=1).astype(jnp.float32)
    x = pred_ref[...]
    p = jax.nn.sigmoid(x)
    logp = jnp.maximum(jnp.log(p), -100.0)
    log1mp = jnp.maximum(jnp.log1p(-p), -100.0)
    loss = -jnp.mean(tvect * logp + (1.0 - tvect) * log1mp)
    out_ref[...] = jnp.reshape(loss, (1, 1))


@jax.jit
def _bce(pred, masks):
    return pl.pallas_call(
        _bce_body,
        out_shape=jax.ShapeDtypeStruct((1, 1), jnp.float32),
    )(pred, masks)


def kernel(pred, target):
    masks = _sc_masks(target.astype(jnp.int32))
    return _bce(pred.astype(jnp.float32), masks)[0, 0]


# speedup vs baseline: 151.3023x; 1.0816x over previous
import functools

import jax
import jax.numpy as jnp
from jax import lax
from jax.experimental import pallas as pl
from jax.experimental.pallas import tpu as pltpu
from jax.experimental.pallas import tpu_sc as plsc

_B = 16
_C = 19
_NC = 2

_R_SC = 256
_R_TC = 512 - _R_SC
_ROWS = 32
_W_ROWS = _R_SC // 2
_NCHUNK = _W_ROWS // _ROWS
_NBUF = 4

_TC_ROWS = 64


def _sc_body(tgt_hbm, out_hbm, buf0, buf1, buf2, buf3, acc_v,
             sem0, sem1, sem2, sem3):
    wid = lax.axis_index("s") * _NC + lax.axis_index("c")
    half = wid // _B
    img = wid % _B
    r0 = half * _W_ROWS

    bufs = (buf0, buf1, buf2, buf3)
    sems = (sem0, sem1, sem2, sem3)
    copies = [None] * _NBUF
    for g in range(min(_NBUF - 1, _NCHUNK)):
        copies[g] = pltpu.async_copy(
            tgt_hbm.at[img, pl.ds(r0 + g * _ROWS, _ROWS)], bufs[g], sems[g])

    accs = tuple(jnp.zeros((16,), jnp.int32) for _ in range(8))
    one = jnp.full((16,), 1, jnp.int32)
    for g in range(_NCHUNK):
        if g + _NBUF - 1 < _NCHUNK:
            copies[(g + _NBUF - 1) % _NBUF] = pltpu.async_copy(
                tgt_hbm.at[img, pl.ds(r0 + (g + _NBUF - 1) * _ROWS, _ROWS)],
                bufs[(g + _NBUF - 1) % _NBUF], sems[(g + _NBUF - 1) % _NBUF])
        copies[g % _NBUF].wait()
        buf = bufs[g % _NBUF]

        def inner(i, a, buf=buf):
            new = list(a)
            for j in range(32):
                new[j % 8] = new[j % 8] | (one << buf[i, pl.ds(j * 16, 16)])
            return tuple(new)

        accs = lax.fori_loop(0, _ROWS, inner, accs)

    acc = accs[0]
    for j in range(1, 8):
        acc = acc | accs[j]
    acc_v[...] = acc
    pltpu.sync_copy(acc_v, out_hbm.at[half, img])


def _sc_masks(tgt):
    mesh = plsc.VectorSubcoreMesh(core_axis_name="c", subcore_axis_name="s")
    f = functools.partial(
        pl.kernel,
        mesh=mesh,
        out_type=jax.ShapeDtypeStruct((2, _B, 16), jnp.int32),
        scratch_types=[
            pltpu.VMEM((_ROWS, 512), jnp.int32),
            pltpu.VMEM((_ROWS, 512), jnp.int32),
            pltpu.VMEM((_ROWS, 512), jnp.int32),
            pltpu.VMEM((_ROWS, 512), jnp.int32),
            pltpu.VMEM((16,), jnp.int32),
            pltpu.SemaphoreType.DMA,
            pltpu.SemaphoreType.DMA,
            pltpu.SemaphoreType.DMA,
            pltpu.SemaphoreType.DMA,
        ],
    )(_sc_body)
    return f(tgt)


def _tc_pres_body(tgt_ref, out_ref, m_acc):
    j = pl.program_id(0)
    t = tgt_ref[...]
    m = jnp.int32(1) << t
    r = _TC_ROWS
    while r > 8:
        m = m[:, : r // 2] | m[:, r // 2:]
        r //= 2
    c = 512
    while c > 128:
        m = m[:, :, : c // 2] | m[:, :, c // 2:]
        c //= 2

    @pl.when(j == 0)
    def _init():
        m_acc[...] = m

    @pl.when(j > 0)
    def _accum():
        m_acc[...] = m_acc[...] | m

    @pl.when(j == pl.num_programs(0) - 1)
    def _emit():
        mm = m_acc[...]
        cls = lax.broadcasted_iota(jnp.int32, (_B, 8, 128, _C), 3)
        bits = (mm[:, :, :, None] >> cls) & 1
        out_ref[...] = jnp.max(bits, axis=(1, 2))


def _tc_pres(tgt):
    return pl.pallas_call(
        _tc_pres_body,
        grid=(_R_TC // _TC_ROWS,),
        in_specs=[pl.BlockSpec(
            (_B, _TC_ROWS, 512),
            lambda j: (0, (_R_SC // _TC_ROWS) + j, 0))],
        out_specs=pl.BlockSpec((_B, _C), lambda j: (0, 0)),
        out_shape=jax.ShapeDtypeStruct((_B, _C), jnp.int32),
        scratch_shapes=[pltpu.VMEM((_B, 8, 128), jnp.int32)],
    )(tgt)


def _bce_body(pred_ref, masks_ref, tv_ref, out_ref):
    m = masks_ref[0] | masks_ref[1]
    cls = lax.broadcasted_iota(jnp.int32, (_B, 16, _C), 2)
    bits = (m[:, :, None] >> cls) & 1
    tvect = jnp.maximum(jnp.max(bits, axis=1), tv_ref[...]).astype(jnp.float32)
    x = pred_ref[...]
    p = jax.nn.sigmoid(x)
    logp = jnp.maximum(jnp.log(p), -100.0)
    log1mp = jnp.maximum(jnp.log1p(-p), -100.0)
    loss = -jnp.mean(tvect * logp + (1.0 - tvect) * log1mp)
    out_ref[...] = jnp.reshape(loss, (1, 1))


def _bce(pred, masks, tv):
    return pl.pallas_call(
        _bce_body,
        out_shape=jax.ShapeDtypeStruct((1, 1), jnp.float32),
    )(pred, masks, tv)


def kernel(pred, target):
    tgt = target.astype(jnp.int32)
    masks = _sc_masks(tgt)
    tv_tc = _tc_pres(tgt)
    return _bce(pred.astype(jnp.float32), masks, tv_tc)[0, 0]


# speedup vs baseline: 155.4413x; 1.0274x over previous
import functools

import jax
import jax.numpy as jnp
from jax import lax
from jax.experimental import pallas as pl
from jax.experimental.pallas import tpu as pltpu
from jax.experimental.pallas import tpu_sc as plsc

_B = 16
_C = 19
_NC = 2

_R_SC = 192
_R_TC = 512 - _R_SC
_ROWS = 32
_W_ROWS = _R_SC // 2
_NCHUNK = _W_ROWS // _ROWS
_NBUF = 4

_TC_ROWS = 64


def _sc_body(tgt_hbm, out_hbm, buf0, buf1, buf2, buf3, acc_v,
             sem0, sem1, sem2, sem3):
    wid = lax.axis_index("s") * _NC + lax.axis_index("c")
    half = wid // _B
    img = wid % _B
    r0 = half * _W_ROWS

    bufs = (buf0, buf1, buf2, buf3)
    sems = (sem0, sem1, sem2, sem3)
    copies = [None] * _NBUF
    for g in range(min(_NBUF - 1, _NCHUNK)):
        copies[g] = pltpu.async_copy(
            tgt_hbm.at[img, pl.ds(r0 + g * _ROWS, _ROWS)], bufs[g], sems[g])

    accs = tuple(jnp.zeros((16,), jnp.int32) for _ in range(8))
    one = jnp.full((16,), 1, jnp.int32)
    for g in range(_NCHUNK):
        if g + _NBUF - 1 < _NCHUNK:
            copies[(g + _NBUF - 1) % _NBUF] = pltpu.async_copy(
                tgt_hbm.at[img, pl.ds(r0 + (g + _NBUF - 1) * _ROWS, _ROWS)],
                bufs[(g + _NBUF - 1) % _NBUF], sems[(g + _NBUF - 1) % _NBUF])
        copies[g % _NBUF].wait()
        buf = bufs[g % _NBUF]

        def inner(i, a, buf=buf):
            new = list(a)
            for j in range(32):
                new[j % 8] = new[j % 8] | (one << buf[i, pl.ds(j * 16, 16)])
            return tuple(new)

        accs = lax.fori_loop(0, _ROWS, inner, accs)

    acc = accs[0]
    for j in range(1, 8):
        acc = acc | accs[j]
    acc_v[...] = acc
    pltpu.sync_copy(acc_v, out_hbm.at[half, img])


def _sc_masks(tgt):
    mesh = plsc.VectorSubcoreMesh(core_axis_name="c", subcore_axis_name="s")
    f = functools.partial(
        pl.kernel,
        mesh=mesh,
        out_type=jax.ShapeDtypeStruct((2, _B, 16), jnp.int32),
        scratch_types=[
            pltpu.VMEM((_ROWS, 512), jnp.int32),
            pltpu.VMEM((_ROWS, 512), jnp.int32),
            pltpu.VMEM((_ROWS, 512), jnp.int32),
            pltpu.VMEM((_ROWS, 512), jnp.int32),
            pltpu.VMEM((16,), jnp.int32),
            pltpu.SemaphoreType.DMA,
            pltpu.SemaphoreType.DMA,
            pltpu.SemaphoreType.DMA,
            pltpu.SemaphoreType.DMA,
        ],
    )(_sc_body)
    return f(tgt)


def _tc_pres_body(tgt_ref, out_ref, m_acc):
    j = pl.program_id(0)
    t = tgt_ref[...]
    m = lax.bitcast_convert_type((t + 127) << 23, jnp.float32).astype(jnp.int32)
    r = _TC_ROWS
    while r > 8:
        m = m[:, : r // 2] | m[:, r // 2:]
        r //= 2
    c = 512
    while c > 128:
        m = m[:, :, : c // 2] | m[:, :, c // 2:]
        c //= 2

    @pl.when(j == 0)
    def _init():
        m_acc[...] = m

    @pl.when(j > 0)
    def _accum():
        m_acc[...] = m_acc[...] | m

    @pl.when(j == pl.num_programs(0) - 1)
    def _emit():
        mm = m_acc[...]
        cls = lax.broadcasted_iota(jnp.int32, (_B, 8, 128, _C), 3)
        bits = (mm[:, :, :, None] >> cls) & 1
        out_ref[...] = jnp.max(bits, axis=(1, 2))


def _tc_pres(tgt):
    return pl.pallas_call(
        _tc_pres_body,
        grid=(_R_TC // _TC_ROWS,),
        in_specs=[pl.BlockSpec(
            (_B, _TC_ROWS, 512),
            lambda j: (0, (_R_SC // _TC_ROWS) + j, 0))],
        out_specs=pl.BlockSpec((_B, _C), lambda j: (0, 0)),
        out_shape=jax.ShapeDtypeStruct((_B, _C), jnp.int32),
        scratch_shapes=[pltpu.VMEM((_B, 8, 128), jnp.int32)],
    )(tgt)


def _bce_body(pred_ref, masks_ref, tv_ref, out_ref):
    m = masks_ref[0] | masks_ref[1]
    cls = lax.broadcasted_iota(jnp.int32, (_B, 16, _C), 2)
    bits = (m[:, :, None] >> cls) & 1
    tvect = jnp.maximum(jnp.max(bits, axis=1), tv_ref[...]).astype(jnp.float32)
    x = pred_ref[...]
    p = jax.nn.sigmoid(x)
    logp = jnp.maximum(jnp.log(p), -100.0)
    log1mp = jnp.maximum(jnp.log1p(-p), -100.0)
    loss = -jnp.mean(tvect * logp + (1.0 - tvect) * log1mp)
    out_ref[...] = jnp.reshape(loss, (1, 1))


def _bce(pred, masks, tv):
    return pl.pallas_call(
        _bce_body,
        out_shape=jax.ShapeDtypeStruct((1, 1), jnp.float32),
    )(pred, masks, tv)


def kernel(pred, target):
    tgt = target.astype(jnp.int32)
    masks = _sc_masks(tgt)
    tv_tc = _tc_pres(tgt)
    return _bce(pred.astype(jnp.float32), masks, tv_tc)[0, 0]
